# baseline (speedup 1.0000x reference)
import numpy as np
import jax
import jax.numpy as jnp
from jax.experimental import pallas as pl

D = 1024
H = 8
DH = 128
NH = 8
S = 52
V = 24
B = 128
BB = 8
RB = BB * S
NBLK = B // BB
BB2 = 4
RB2 = BB2 * S
NBLK2 = B // BB2
NBH = S // 2

HI = jax.lax.Precision.HIGHEST
DE = jax.lax.Precision.DEFAULT


def _make_pe():
    pos = np.arange(S)[:, None].astype(np.float64)
    i = np.arange(D)[None, :]
    angle = pos / np.power(10000.0, (2 * (i // 2)) / float(D))
    pe = np.zeros((S, D))
    pe[:, 0::2] = np.sin(angle[:, 0::2])
    pe[:, 1::2] = np.cos(angle[:, 1::2])
    return pe.astype(np.float32)


_PE_NP = _make_pe()


def _dot(a, b, prec):
    return jax.lax.dot_general(a, b, (((1,), (0,)), ((), ())),
                               preferred_element_type=jnp.float32,
                               precision=prec)


def _dott(a, b, prec):
    return jax.lax.dot_general(a, b, (((1,), (1,)), ((), ())),
                               preferred_element_type=jnp.float32,
                               precision=prec)


def _dotc0(a, b, prec):
    return jax.lax.dot_general(a, b, (((0,), (0,)), ((), ())),
                               preferred_element_type=jnp.float32,
                               precision=prec)


def _ln(h, g, b):
    mu = jnp.mean(h, axis=1, keepdims=True)
    var = jnp.mean((h - mu) ** 2, axis=1, keepdims=True)
    return (h - mu) / jnp.sqrt(var + 1e-5) * g + b


def _qkv_buckets(hn, rot_ref, wqk_ref, wv_ref, rows):
    qk_all = _dot(hn, wqk_ref[:], DE)
    v_all = _dot(hn, wv_ref[:], DE)
    qk_t = jnp.transpose(qk_all)
    ohs = []
    for h in range(H):
        rott = _dot(rot_ref[:], qk_t[h * DH:(h + 1) * DH, :], DE)
        r3 = rott.reshape(NH, 64, rows)
        mx = jnp.max(r3, axis=1, keepdims=True)
        io3 = jax.lax.broadcasted_iota(jnp.int32, (NH, 64, rows), 1)
        idx = jnp.min(jnp.where(r3 >= mx, io3, 64), axis=1, keepdims=True)
        ohs.append((io3 == idx).astype(jnp.float32).reshape(NH * 64, rows))
    return qk_all, v_all, ohs


def _attn_embed_kernel(xl_ref, xc_ref, emb_ref, pe_ref, g_ref, b_ref,
                       wqk_ref, wv_ref, wo_ref, bo_ref, rot_ref, out_ref):
    xc = xc_ref[0]
    xl = xl_ref[0]
    onehot = (xc == jax.lax.broadcasted_iota(jnp.int32, (RB, V), 1))
    hin = _dot(onehot.astype(jnp.float32), emb_ref[:], HI) + pe_ref[:]
    mk_col = (xc != 0).astype(jnp.float32)
    hn = _ln(hin, g_ref[0], b_ref[0])

    ri = jax.lax.broadcasted_iota(jnp.int32, (RB, RB), 0)
    ci = jax.lax.broadcasted_iota(jnp.int32, (RB, RB), 1)
    bdiag = (ri // S) == (ci // S)
    eye = ri == ci
    keymask = jnp.broadcast_to(xl != 0, (RB, RB))

    qk_all, v_all, ohs = _qkv_buckets(hn, rot_ref, wqk_ref, wv_ref, RB)
    outs = []
    for h in range(H):
        qk = qk_all[:, h * DH:(h + 1) * DH]
        cnt = _dotc0(ohs[h], ohs[h], DE)
        nrm = jnp.sqrt(jnp.sum(qk * qk, axis=1, keepdims=True))
        kk = qk / (nrm + 1e-8)
        s = _dott(qk, kk, DE) / np.float32(np.sqrt(float(DH)))
        s = jnp.where((cnt > 0.5) & bdiag & keymask, s, -1e9)
        s = jnp.where(eye, np.float32(-1e5), s)
        m = jnp.max(s, axis=1, keepdims=True)
        e = jnp.exp(s - m)
        p = e / jnp.sum(e, axis=1, keepdims=True)
        outs.append(_dot(p, v_all[:, h * DH:(h + 1) * DH], DE))
    att = jnp.concatenate(outs, axis=1)
    o = _dot(att, wo_ref[:], DE) + bo_ref[0]
    out_ref[:] = hin + mk_col * o


def _ff_kernel(hin_ref, g_ref, b_ref, w1_ref, b1_ref, w2_ref, b2_ref,
               out_ref):
    hin = hin_ref[:]
    hn = _ln(hin, g_ref[0], b_ref[0])
    a = _dot(hn, w1_ref[:], DE) + b1_ref[0]
    gg = jax.nn.gelu(a)
    out_ref[:] = hin + _dot(gg, w2_ref[:], DE) + b2_ref[0]


def _attn2ff2_kernel(xl_ref, hq_ref, hin_ref, g_ref, b_ref,
                     wqk_ref, wv_ref, wo_ref, bo_ref, rot_ref,
                     g2_ref, b2_ref, w1_ref, b1_ref, w2_ref, b2b_ref,
                     out_ref):
    xl = xl_ref[0]
    hin = hin_ref[:]
    hq = hq_ref[0]
    hn = _ln(hin, g_ref[0], b_ref[0])

    qk_all, v_all, ohs = _qkv_buckets(hn, rot_ref, wqk_ref, wv_ref, RB2)

    hqn = _ln(hq, g_ref[0], b_ref[0])
    qk0_all = _dot(hqn, wqk_ref[:], DE)
    ri8 = jax.lax.broadcasted_iota(jnp.int32, (BB2, RB2), 0)
    ci8 = jax.lax.broadcasted_iota(jnp.int32, (BB2, RB2), 1)
    sel = (ci8 == ri8 * S).astype(jnp.float32)
    bdiag0 = ri8 == (ci8 // S)
    eye0 = ci8 == ri8 * S
    keymask0 = jnp.broadcast_to(xl != 0, (BB2, RB2))
    mk0 = _dot(sel, (jnp.transpose(xl) != 0).astype(jnp.float32), DE)

    outs = []
    for h in range(H):
        qk = qk_all[:, h * DH:(h + 1) * DH]
        qk0 = qk0_all[:, h * DH:(h + 1) * DH]
        ohq = _dott(sel, ohs[h], DE)
        cnt0 = _dot(ohq, ohs[h], DE)
        nrm = jnp.sqrt(jnp.sum(qk * qk, axis=1, keepdims=True))
        kk = qk / (nrm + 1e-8)
        s = _dott(qk0, kk, DE) / np.float32(np.sqrt(float(DH)))
        s = jnp.where((cnt0 > 0.5) & bdiag0 & keymask0, s, -1e9)
        s = jnp.where(eye0, np.float32(-1e5), s)
        m = jnp.max(s, axis=1, keepdims=True)
        e = jnp.exp(s - m)
        p = e / jnp.sum(e, axis=1, keepdims=True)
        outs.append(_dot(p, v_all[:, h * DH:(h + 1) * DH], DE))
    att = jnp.concatenate(outs, axis=1)
    o = _dot(att, wo_ref[:], DE) + bo_ref[0]
    h2 = hq + mk0 * o

    hn2 = _ln(h2, g2_ref[0], b2_ref[0])
    a = _dot(hn2, w1_ref[:], DE) + b1_ref[0]
    gg = jax.nn.gelu(a)
    out_ref[0] = h2 + _dot(gg, w2_ref[:], DE) + b2b_ref[0]


def _whole(arr):
    nd = arr.ndim
    return pl.BlockSpec(arr.shape, lambda i, _nd=nd: (0,) * _nd)


def kernel(x, params):
    x = x.astype(jnp.int32)
    l1, l2 = params['layers']
    emb = params['token_emb']
    pet = jnp.asarray(np.tile(_PE_NP, (BB, 1)))
    rot3 = jax.random.normal(jax.random.key(42), (DH, NH, NBH),
                             dtype=jnp.float32)
    rt = jnp.transpose(rot3, (1, 2, 0))
    rt = jnp.concatenate([rt, -rt], axis=1)
    rt = jnp.pad(rt, ((0, 0), (0, 64 - S), (0, 0)))
    rot = rt.reshape(NH * 64, DH)

    xf = x.reshape(-1)

    def row2(a):
        return a.reshape(1, -1)

    xl = xf.reshape(NBLK, 1, RB)
    xc = xf.reshape(NBLK, RB, 1)
    io_spec = pl.BlockSpec((RB, D), lambda i: (i, 0))
    args = [xl, xc, emb, pet, row2(l1['ln1_g']), row2(l1['ln1_b']),
            l1['wqk'], l1['wv'], l1['wo'], row2(l1['bo']), rot]
    in_specs = [pl.BlockSpec((1, 1, RB), lambda i: (i, 0, 0)),
                pl.BlockSpec((1, RB, 1), lambda i: (i, 0, 0))] + \
               [_whole(a) for a in args[2:]]
    h = pl.pallas_call(
        _attn_embed_kernel, grid=(NBLK,), in_specs=in_specs,
        out_specs=io_spec,
        out_shape=jax.ShapeDtypeStruct((B * S, D), jnp.float32),
    )(*args)

    ff_spec = pl.BlockSpec((RB2, D), lambda i: (i, 0))
    args = [h, row2(l1['ln2_g']), row2(l1['ln2_b']),
            l1['w1'], row2(l1['b1']), l1['w2'], row2(l1['b2'])]
    h = pl.pallas_call(
        _ff_kernel, grid=(B * S // RB2,),
        in_specs=[ff_spec] + [_whole(a) for a in args[1:]],
        out_specs=ff_spec,
        out_shape=jax.ShapeDtypeStruct((B * S, D), jnp.float32),
    )(*args)

    hq = h.reshape(B, S, D)[:, 0, :].reshape(NBLK2, BB2, D)
    xl2 = xf.reshape(NBLK2, 1, RB2)
    args = [xl2, hq, h, row2(l2['ln1_g']), row2(l2['ln1_b']),
            l2['wqk'], l2['wv'], l2['wo'], row2(l2['bo']), rot,
            row2(l2['ln2_g']), row2(l2['ln2_b']),
            l2['w1'], row2(l2['b1']), l2['w2'], row2(l2['b2'])]
    in_specs = [pl.BlockSpec((1, 1, RB2), lambda i: (i, 0, 0)),
                pl.BlockSpec((1, BB2, D), lambda i: (i, 0, 0)),
                pl.BlockSpec((RB2, D), lambda i: (i, 0))] + \
               [_whole(a) for a in args[3:]]
    out = pl.pallas_call(
        _attn2ff2_kernel, grid=(NBLK2,), in_specs=in_specs,
        out_specs=pl.BlockSpec((1, BB2, D), lambda i: (i, 0, 0)),
        out_shape=jax.ShapeDtypeStruct((NBLK2, BB2, D), jnp.float32),
    )(*args)
    return out.reshape(B, D)


# speedup vs baseline: 1.1926x; 1.1926x over previous
import numpy as np
import jax
import jax.numpy as jnp
from jax.experimental import pallas as pl

D = 1024
H = 8
DH = 128
NH = 8
S = 52
V = 24
B = 128
BB = 8
RB = BB * S
NBLK = B // BB
NBH = S // 2

HI = jax.lax.Precision.HIGHEST
DE = jax.lax.Precision.DEFAULT


def _make_pe():
    pos = np.arange(S)[:, None].astype(np.float64)
    i = np.arange(D)[None, :]
    angle = pos / np.power(10000.0, (2 * (i // 2)) / float(D))
    pe = np.zeros((S, D))
    pe[:, 0::2] = np.sin(angle[:, 0::2])
    pe[:, 1::2] = np.cos(angle[:, 1::2])
    return pe.astype(np.float32)


_PE_NP = _make_pe()


def _dot(a, b, prec):
    return jax.lax.dot_general(a, b, (((1,), (0,)), ((), ())),
                               preferred_element_type=jnp.float32,
                               precision=prec)


def _dott(a, b, prec):
    return jax.lax.dot_general(a, b, (((1,), (1,)), ((), ())),
                               preferred_element_type=jnp.float32,
                               precision=prec)


def _dotc0(a, b, prec):
    return jax.lax.dot_general(a, b, (((0,), (0,)), ((), ())),
                               preferred_element_type=jnp.float32,
                               precision=prec)


def _ln(h, g, b):
    mu = jnp.mean(h, axis=1, keepdims=True)
    var = jnp.mean((h - mu) ** 2, axis=1, keepdims=True)
    return (h - mu) / jnp.sqrt(var + 1e-5) * g + b


def _bucket_oh(qk_t, rot_ref, h):
    rows = qk_t.shape[1]
    rott = _dot(rot_ref[:], qk_t[h * DH:(h + 1) * DH, :], DE)
    r3 = rott.reshape(NH, 64, rows)
    mx = jnp.max(r3, axis=1, keepdims=True)
    io3 = jax.lax.broadcasted_iota(jnp.int32, (NH, 64, rows), 1)
    idx = jnp.min(jnp.where(r3 >= mx, io3, 64), axis=1, keepdims=True)
    return (io3 == idx).astype(jnp.float32).reshape(NH * 64, rows)


def _attn_embed_kernel(xl_ref, xc_ref, emb_ref, pe_ref, g_ref, b_ref,
                       wqk_ref, wv_ref, wo_ref, bo_ref, rot_ref, out_ref):
    xc = xc_ref[0]
    xl = xl_ref[0]
    onehot = (xc == jax.lax.broadcasted_iota(jnp.int32, (RB, V), 1))
    hin = _dot(onehot.astype(jnp.float32), emb_ref[:], HI) + pe_ref[:]
    mk_col = (xc != 0).astype(jnp.float32)
    hn = _ln(hin, g_ref[0], b_ref[0])

    ri = jax.lax.broadcasted_iota(jnp.int32, (RB, RB), 0)
    ci = jax.lax.broadcasted_iota(jnp.int32, (RB, RB), 1)
    eye = ri == ci
    bd2 = ((ri // S) == (ci // S)) & (xl != 0)

    qk_all = _dot(hn, wqk_ref[:], DE)
    v_all = _dot(hn, wv_ref[:], DE)
    qk_t = jnp.transpose(qk_all)

    outs = []
    for h in range(H):
        qk = qk_all[:, h * DH:(h + 1) * DH]
        oh = _bucket_oh(qk_t, rot_ref, h)
        cnt = _dotc0(oh, oh, DE)
        nrm = jnp.sqrt(jnp.sum(qk * qk, axis=1, keepdims=True))
        kk = qk / (nrm + 1e-8)
        s = _dott(qk, kk, DE) / np.float32(np.sqrt(float(DH)))
        s = jnp.where((cnt > 0.5) & bd2, s, -1e9)
        s = jnp.where(eye, np.float32(-1e5), s)
        m = jnp.max(s, axis=1, keepdims=True)
        e = jnp.exp(s - m)
        p = e / jnp.sum(e, axis=1, keepdims=True)
        outs.append(_dot(p, v_all[:, h * DH:(h + 1) * DH], DE))
    att = jnp.concatenate(outs, axis=1)
    o = _dot(att, wo_ref[:], DE) + bo_ref[0]
    out_ref[:] = hin + mk_col * o


def _ff_kernel(hin_ref, g_ref, b_ref, w1_ref, b1_ref, w2_ref, b2_ref,
               out_ref):
    hin = hin_ref[:]
    hn = _ln(hin, g_ref[0], b_ref[0])
    a = _dot(hn, w1_ref[:], DE) + b1_ref[0]
    gg = jax.nn.gelu(a)
    out_ref[:] = hin + _dot(gg, w2_ref[:], DE) + b2_ref[0]


def _attn2_kernel(xl_ref, hq_ref, hin_ref, g_ref, b_ref,
                  wqk_ref, wv_ref, wo_ref, bo_ref, rot_ref, out_ref):
    xl = xl_ref[0]
    hin = hin_ref[:]
    hq = hq_ref[0]
    hn = _ln(hin, g_ref[0], b_ref[0])

    qk_all = _dot(hn, wqk_ref[:], DE)
    v_all = _dot(hn, wv_ref[:], DE)
    qk_t = jnp.transpose(qk_all)

    hqn = _ln(hq, g_ref[0], b_ref[0])
    qk0_all = _dot(hqn, wqk_ref[:], DE)
    ri8 = jax.lax.broadcasted_iota(jnp.int32, (BB, RB), 0)
    ci8 = jax.lax.broadcasted_iota(jnp.int32, (BB, RB), 1)
    sel = (ci8 == ri8 * S).astype(jnp.float32)
    eye0 = ci8 == ri8 * S
    bd20 = (ri8 == (ci8 // S)) & (xl != 0)
    mk0 = _dot(sel, (jnp.transpose(xl) != 0).astype(jnp.float32), DE)

    outs = []
    for h in range(H):
        qk = qk_all[:, h * DH:(h + 1) * DH]
        qk0 = qk0_all[:, h * DH:(h + 1) * DH]
        oh = _bucket_oh(qk_t, rot_ref, h)
        ohq = _dott(sel, oh, DE)
        cnt0 = _dot(ohq, oh, DE)
        nrm = jnp.sqrt(jnp.sum(qk * qk, axis=1, keepdims=True))
        kk = qk / (nrm + 1e-8)
        s = _dott(qk0, kk, DE) / np.float32(np.sqrt(float(DH)))
        s = jnp.where((cnt0 > 0.5) & bd20, s, -1e9)
        s = jnp.where(eye0, np.float32(-1e5), s)
        m = jnp.max(s, axis=1, keepdims=True)
        e = jnp.exp(s - m)
        p = e / jnp.sum(e, axis=1, keepdims=True)
        outs.append(_dot(p, v_all[:, h * DH:(h + 1) * DH], DE))
    att = jnp.concatenate(outs, axis=1)
    o = _dot(att, wo_ref[:], DE) + bo_ref[0]
    out_ref[0] = hq + mk0 * o


def _whole(arr):
    nd = arr.ndim
    return pl.BlockSpec(arr.shape, lambda i, _nd=nd: (0,) * _nd)


def kernel(x, params):
    x = x.astype(jnp.int32)
    l1, l2 = params['layers']
    emb = params['token_emb']
    pet = jnp.asarray(np.tile(_PE_NP, (BB, 1)))
    rot3 = jax.random.normal(jax.random.key(42), (DH, NH, NBH),
                             dtype=jnp.float32)
    rt = jnp.transpose(rot3, (1, 2, 0))
    rt = jnp.concatenate([rt, -rt], axis=1)
    rt = jnp.pad(rt, ((0, 0), (0, 64 - S), (0, 0)))
    rot = rt.reshape(NH * 64, DH)

    xf = x.reshape(-1)
    xl = xf.reshape(NBLK, 1, RB)
    xc = xf.reshape(NBLK, RB, 1)

    def row2(a):
        return a.reshape(1, -1)

    io_spec = pl.BlockSpec((RB, D), lambda i: (i, 0))
    xl_spec = pl.BlockSpec((1, 1, RB), lambda i: (i, 0, 0))
    args = [xl, xc, emb, pet, row2(l1['ln1_g']), row2(l1['ln1_b']),
            l1['wqk'], l1['wv'], l1['wo'], row2(l1['bo']), rot]
    in_specs = [xl_spec,
                pl.BlockSpec((1, RB, 1), lambda i: (i, 0, 0))] + \
               [_whole(a) for a in args[2:]]
    h = pl.pallas_call(
        _attn_embed_kernel, grid=(NBLK,), in_specs=in_specs,
        out_specs=io_spec,
        out_shape=jax.ShapeDtypeStruct((B * S, D), jnp.float32),
    )(*args)

    ff_spec = pl.BlockSpec((RB // 2, D), lambda i: (i, 0))
    args = [h, row2(l1['ln2_g']), row2(l1['ln2_b']),
            l1['w1'], row2(l1['b1']), l1['w2'], row2(l1['b2'])]
    h = pl.pallas_call(
        _ff_kernel, grid=(B * S // (RB // 2),),
        in_specs=[ff_spec] + [_whole(a) for a in args[1:]],
        out_specs=ff_spec,
        out_shape=jax.ShapeDtypeStruct((B * S, D), jnp.float32),
    )(*args)

    hq = h.reshape(B, S, D)[:, 0, :].reshape(NBLK, BB, D)
    args = [xl, hq, h, row2(l2['ln1_g']), row2(l2['ln1_b']),
            l2['wqk'], l2['wv'], l2['wo'], row2(l2['bo']), rot]
    in_specs = [xl_spec,
                pl.BlockSpec((1, BB, D), lambda i: (i, 0, 0)),
                io_spec] + [_whole(a) for a in args[3:]]
    h2 = pl.pallas_call(
        _attn2_kernel, grid=(NBLK,), in_specs=in_specs,
        out_specs=pl.BlockSpec((1, BB, D), lambda i: (i, 0, 0)),
        out_shape=jax.ShapeDtypeStruct((NBLK, BB, D), jnp.float32),
    )(*args).reshape(B, D)

    args = [h2, row2(l2['ln2_g']), row2(l2['ln2_b']),
            l2['w1'], row2(l2['b1']), l2['w2'], row2(l2['b2'])]
    out = pl.pallas_call(
        _ff_kernel, grid=(1,),
        in_specs=[_whole(h2)] + [_whole(a) for a in args[1:]],
        out_specs=_whole(h2),
        out_shape=jax.ShapeDtypeStruct((B, D), jnp.float32),
    )(*args)
    return out


# speedup vs baseline: 1.1987x; 1.0051x over previous
import numpy as np
import jax
import jax.numpy as jnp
from jax.experimental import pallas as pl

D = 1024
H = 8
DH = 128
NH = 8
S = 52
V = 24
B = 128
BB = 8
RB = BB * S
NBLK = B // BB
NBH = S // 2

HI = jax.lax.Precision.HIGHEST
DE = jax.lax.Precision.DEFAULT


def _make_pe():
    pos = np.arange(S)[:, None].astype(np.float64)
    i = np.arange(D)[None, :]
    angle = pos / np.power(10000.0, (2 * (i // 2)) / float(D))
    pe = np.zeros((S, D))
    pe[:, 0::2] = np.sin(angle[:, 0::2])
    pe[:, 1::2] = np.cos(angle[:, 1::2])
    return pe.astype(np.float32)


_PE_NP = _make_pe()


def _dot(a, b, prec):
    return jax.lax.dot_general(a, b, (((1,), (0,)), ((), ())),
                               preferred_element_type=jnp.float32,
                               precision=prec)


def _dott(a, b, prec):
    return jax.lax.dot_general(a, b, (((1,), (1,)), ((), ())),
                               preferred_element_type=jnp.float32,
                               precision=prec)


def _dotc0(a, b, prec):
    return jax.lax.dot_general(a, b, (((0,), (0,)), ((), ())),
                               preferred_element_type=jnp.float32,
                               precision=prec)


def _ln(h, g, b):
    mu = jnp.mean(h, axis=1, keepdims=True)
    var = jnp.mean((h - mu) ** 2, axis=1, keepdims=True)
    return (h - mu) / jnp.sqrt(var + 1e-5) * g + b


def _bucket_oh(qk_t, rot_ref, h, io3):
    rows = qk_t.shape[1]
    rott = _dot(rot_ref[:], qk_t[h * DH:(h + 1) * DH, :], DE)
    r3 = rott.reshape(NH, 64, rows)
    mx = jnp.max(r3, axis=1, keepdims=True)
    idx = jnp.min(jnp.where(r3 >= mx, io3, 64), axis=1, keepdims=True)
    return (io3 == idx).astype(jnp.float32).reshape(NH * 64, rows)


def _attn_embed_kernel(xl_ref, xc_ref, emb_ref, pe_ref, g_ref, b_ref,
                       wqk_ref, wv_ref, wo_ref, bo_ref, rot_ref, out_ref):
    xc = xc_ref[0]
    xl = xl_ref[0]
    onehot = (xc == jax.lax.broadcasted_iota(jnp.int32, (RB, V), 1))
    hin = _dot(onehot.astype(jnp.float32), emb_ref[:], HI) + pe_ref[:]
    mk_col = (xc != 0).astype(jnp.float32)
    hn = _ln(hin, g_ref[0], b_ref[0])

    ri = jax.lax.broadcasted_iota(jnp.int32, (RB, RB), 0)
    ci = jax.lax.broadcasted_iota(jnp.int32, (RB, RB), 1)
    eye = ri == ci
    bd2 = ((ri // S) == (ci // S)) & (xl != 0)

    qk_all = _dot(hn, wqk_ref[:], DE)
    v_all = _dot(hn, wv_ref[:], DE)
    qk_t = jnp.transpose(qk_all)

    io3 = jax.lax.broadcasted_iota(jnp.int32, (NH, 64, RB), 1)
    isq = np.float32(1.0 / np.sqrt(float(DH)))
    outs = []
    for h in range(H):
        qk = qk_all[:, h * DH:(h + 1) * DH]
        oh = _bucket_oh(qk_t, rot_ref, h, io3)
        cnt = _dotc0(oh, oh, DE)
        nrm = jnp.sqrt(jnp.sum(qk * qk, axis=1, keepdims=True))
        kk = qk * (isq / (nrm + 1e-8))
        s = _dott(qk, kk, DE)
        s = jnp.where((cnt > 0.5) & bd2, s, -1e9)
        s = jnp.where(eye, np.float32(-1e5), s)
        m = jnp.max(s, axis=1, keepdims=True)
        e = jnp.exp(s - m)
        p = e * (1.0 / jnp.sum(e, axis=1, keepdims=True))
        outs.append(_dot(p, v_all[:, h * DH:(h + 1) * DH], DE))
    att = jnp.concatenate(outs, axis=1)
    o = _dot(att, wo_ref[:], DE) + bo_ref[0]
    out_ref[:] = hin + mk_col * o


def _ff_kernel(hin_ref, g_ref, b_ref, w1_ref, b1_ref, w2_ref, b2_ref,
               out_ref):
    hin = hin_ref[:]
    hn = _ln(hin, g_ref[0], b_ref[0])
    a = _dot(hn, w1_ref[:], DE) + b1_ref[0]
    gg = jax.nn.gelu(a)
    out_ref[:] = hin + _dot(gg, w2_ref[:], DE) + b2_ref[0]


def _attn2_kernel(xl_ref, hq_ref, hin_ref, g_ref, b_ref,
                  wqk_ref, wv_ref, wo_ref, bo_ref, rot_ref, out_ref):
    xl = xl_ref[0]
    hin = hin_ref[:]
    hq = hq_ref[0]
    hn = _ln(hin, g_ref[0], b_ref[0])

    qk_all = _dot(hn, wqk_ref[:], DE)
    v_all = _dot(hn, wv_ref[:], DE)
    qk_t = jnp.transpose(qk_all)

    hqn = _ln(hq, g_ref[0], b_ref[0])
    qk0_all = _dot(hqn, wqk_ref[:], DE)
    ri8 = jax.lax.broadcasted_iota(jnp.int32, (BB, RB), 0)
    ci8 = jax.lax.broadcasted_iota(jnp.int32, (BB, RB), 1)
    sel = (ci8 == ri8 * S).astype(jnp.float32)
    eye0 = ci8 == ri8 * S
    bd20 = (ri8 == (ci8 // S)) & (xl != 0)
    mk0 = _dot(sel, (jnp.transpose(xl) != 0).astype(jnp.float32), DE)

    io3 = jax.lax.broadcasted_iota(jnp.int32, (NH, 64, RB), 1)
    isq = np.float32(1.0 / np.sqrt(float(DH)))
    outs = []
    for h in range(H):
        qk = qk_all[:, h * DH:(h + 1) * DH]
        qk0 = qk0_all[:, h * DH:(h + 1) * DH]
        oh = _bucket_oh(qk_t, rot_ref, h, io3)
        ohq = _dott(sel, oh, DE)
        cnt0 = _dot(ohq, oh, DE)
        nrm = jnp.sqrt(jnp.sum(qk * qk, axis=1, keepdims=True))
        kk = qk * (isq / (nrm + 1e-8))
        s = _dott(qk0, kk, DE)
        s = jnp.where((cnt0 > 0.5) & bd20, s, -1e9)
        s = jnp.where(eye0, np.float32(-1e5), s)
        m = jnp.max(s, axis=1, keepdims=True)
        e = jnp.exp(s - m)
        p = e * (1.0 / jnp.sum(e, axis=1, keepdims=True))
        outs.append(_dot(p, v_all[:, h * DH:(h + 1) * DH], DE))
    att = jnp.concatenate(outs, axis=1)
    o = _dot(att, wo_ref[:], DE) + bo_ref[0]
    out_ref[0] = hq + mk0 * o


def _whole(arr):
    nd = arr.ndim
    return pl.BlockSpec(arr.shape, lambda i, _nd=nd: (0,) * _nd)


def kernel(x, params):
    x = x.astype(jnp.int32)
    l1, l2 = params['layers']
    emb = params['token_emb']
    pet = jnp.asarray(np.tile(_PE_NP, (BB, 1)))
    rot3 = jax.random.normal(jax.random.key(42), (DH, NH, NBH),
                             dtype=jnp.float32)
    rt = jnp.transpose(rot3, (1, 2, 0))
    rt = jnp.concatenate([rt, -rt], axis=1)
    rt = jnp.pad(rt, ((0, 0), (0, 64 - S), (0, 0)))
    rot = rt.reshape(NH * 64, DH)

    xf = x.reshape(-1)
    xl = xf.reshape(NBLK, 1, RB)
    xc = xf.reshape(NBLK, RB, 1)

    def row2(a):
        return a.reshape(1, -1)

    io_spec = pl.BlockSpec((RB, D), lambda i: (i, 0))
    xl_spec = pl.BlockSpec((1, 1, RB), lambda i: (i, 0, 0))
    args = [xl, xc, emb, pet, row2(l1['ln1_g']), row2(l1['ln1_b']),
            l1['wqk'], l1['wv'], l1['wo'], row2(l1['bo']), rot]
    in_specs = [xl_spec,
                pl.BlockSpec((1, RB, 1), lambda i: (i, 0, 0))] + \
               [_whole(a) for a in args[2:]]
    h = pl.pallas_call(
        _attn_embed_kernel, grid=(NBLK,), in_specs=in_specs,
        out_specs=io_spec,
        out_shape=jax.ShapeDtypeStruct((B * S, D), jnp.float32),
    )(*args)

    ff_spec = pl.BlockSpec((RB // 2, D), lambda i: (i, 0))
    args = [h, row2(l1['ln2_g']), row2(l1['ln2_b']),
            l1['w1'], row2(l1['b1']), l1['w2'], row2(l1['b2'])]
    h = pl.pallas_call(
        _ff_kernel, grid=(B * S // (RB // 2),),
        in_specs=[ff_spec] + [_whole(a) for a in args[1:]],
        out_specs=ff_spec,
        out_shape=jax.ShapeDtypeStruct((B * S, D), jnp.float32),
    )(*args)

    hq = h.reshape(B, S, D)[:, 0, :].reshape(NBLK, BB, D)
    args = [xl, hq, h, row2(l2['ln1_g']), row2(l2['ln1_b']),
            l2['wqk'], l2['wv'], l2['wo'], row2(l2['bo']), rot]
    in_specs = [xl_spec,
                pl.BlockSpec((1, BB, D), lambda i: (i, 0, 0)),
                io_spec] + [_whole(a) for a in args[3:]]
    h2 = pl.pallas_call(
        _attn2_kernel, grid=(NBLK,), in_specs=in_specs,
        out_specs=pl.BlockSpec((1, BB, D), lambda i: (i, 0, 0)),
        out_shape=jax.ShapeDtypeStruct((NBLK, BB, D), jnp.float32),
    )(*args).reshape(B, D)

    args = [h2, row2(l2['ln2_g']), row2(l2['ln2_b']),
            l2['w1'], row2(l2['b1']), l2['w2'], row2(l2['b2'])]
    out = pl.pallas_call(
        _ff_kernel, grid=(1,),
        in_specs=[_whole(h2)] + [_whole(a) for a in args[1:]],
        out_specs=_whole(h2),
        out_shape=jax.ShapeDtypeStruct((B, D), jnp.float32),
    )(*args)
    return out
